# baseline (speedup 1.0000x reference)
import functools

import jax
import jax.numpy as jnp
from jax import lax
from jax.experimental import pallas as pl
from jax.experimental.pallas import tpu as pltpu
from jax.experimental.pallas import tpu_sc as plsc

N_NODES = 100000
D = 64
N_CELLS = 1024
B = 16384


TB = 512
N_TILES = B // TB


def _tc_body(emb_ref, cent_ref, times_ref, upd_ref):
  emb = emb_ref[...]
  cent = cent_ref[...]
  e2 = jnp.sum(emb * emb, axis=1, keepdims=True)
  c2 = jnp.sum(cent * cent, axis=1)[None, :]
  dot = jax.lax.dot_general(
      emb, cent, (((1,), (1,)), ((), ())),
      preferred_element_type=jnp.float32)
  d2 = e2 - 2.0 * dot + c2
  minval = jnp.min(d2, axis=1, keepdims=True)
  iota = lax.broadcasted_iota(jnp.int32, (TB, N_CELLS), 1)
  idx = jnp.min(jnp.where(d2 == minval, iota, N_CELLS), axis=1)
  onehot = (iota == idx[:, None]).astype(jnp.float32)
  gathered = jax.lax.dot_general(
      onehot, cent, (((1,), (0,)), ((), ())),
      preferred_element_type=jnp.float32,
      precision=jax.lax.Precision.HIGHEST)
  t = times_ref[0, 0, :]
  w = jax.nn.sigmoid(t)[:, None]
  upd_ref[...] = w * emb + (1.0 - w) * gathered


def _tc_stage(embeddings, map_centers, times):
  times3 = times.reshape(N_TILES, 1, TB)
  return pl.pallas_call(
      _tc_body,
      grid=(N_TILES,),
      in_specs=[
          pl.BlockSpec((TB, D), lambda i: (i, 0)),
          pl.BlockSpec((N_CELLS, D), lambda i: (0, 0)),
          pl.BlockSpec((1, 1, TB), lambda i: (i, 0, 0)),
      ],
      out_specs=pl.BlockSpec((TB, D), lambda i: (i, 0)),
      out_shape=jax.ShapeDtypeStruct((B, D), jnp.float32),
  )(embeddings, map_centers, times3)


NC = 2
NS = 16
NW = NC * NS
RANGE = 3128
LAST = N_NODES - (NW - 1) * RANGE
TPAD = ((RANGE + 15) // 16) * 16
CHUNK = 128
NCHUNKS = (RANGE + CHUNK - 1) // CHUNK
IDV = B // 16


def _sc_body(mem_hbm, ids_hbm, upd_hbm, out_hbm,
             ids_v, table_v, wb_v, wn_v, rows_v, sem, csem):
  wid = lax.axis_index("s") * NC + lax.axis_index("c")
  base = pl.multiple_of(wid * RANGE, 8)
  is_last = wid == NW - 1

  copy = pltpu.make_async_copy(
      mem_hbm.at[pl.ds(base, RANGE)], out_hbm.at[pl.ds(base, RANGE)], csem)
  copy_last = pltpu.make_async_copy(
      mem_hbm.at[pl.ds(base, LAST)], out_hbm.at[pl.ds(base, LAST)], csem)

  @pl.when(jnp.logical_not(is_last))
  def _():
    copy.start()

  @pl.when(is_last)
  def _():
    copy_last.start()

  def copy_wait():
    @pl.when(jnp.logical_not(is_last))
    def _():
      copy.wait()

    @pl.when(is_last)
    def _():
      copy_last.wait()

  pltpu.sync_copy(ids_hbm, ids_v)

  lanes = lax.iota(jnp.int32, 16)

  def init_body(k, _):
    table_v[pl.ds(k * 16, 16)] = jnp.full((16,), -1, jnp.int32)
    return 0
  lax.fori_loop(0, TPAD // 16, init_body, 0)

  def scan_body(i, _):
    ids16 = ids_v[pl.ds(i * 16, 16)]
    local = ids16 - base
    msk = (local >= 0) & (local < RANGE)
    ivec = lanes + i * 16
    plsc.store_scatter(table_v, [local], ivec, mask=msk)
    return 0
  lax.fori_loop(0, IDV, scan_body, 0)
  plsc.subcore_barrier()

  def compact_body(k, cnt):
    ent = table_v[pl.ds(k * 16, 16)]
    m = ent >= 0
    pos = cnt + plsc.cumsum(m.astype(jnp.int32)) - 1
    node = lanes + (k * 16 + base)
    plsc.store_scatter(wb_v, [pos // CHUNK, pos % CHUNK], ent, mask=m)
    plsc.store_scatter(wn_v, [pos // CHUNK, pos % CHUNK], node, mask=m)
    return cnt + jnp.sum(m.astype(jnp.int32))
  m_w = lax.fori_loop(0, TPAD // 16, compact_body, 0)
  plsc.subcore_barrier()

  @pl.when(m_w > 0)
  def _do_scatter():
    wb16 = wb_v[0, pl.ds(0, 16)]
    wn16 = wn_v[0, pl.ds(0, 16)]
    sel0 = (lanes == 0).astype(jnp.int32)
    wb0 = jnp.sum(wb16 * sel0)
    wn0 = jnp.sum(wn16 * sel0)
    n_chunks = (m_w + CHUNK - 1) // CHUNK
    padded = n_chunks * CHUNK

    def pad_body(j, _):
      pos = m_w + j * 16 + lanes
      pm = pos < padded
      plsc.store_scatter(wb_v, [pos // CHUNK, pos % CHUNK],
                         jnp.full((16,), 0, jnp.int32) + wb0, mask=pm)
      plsc.store_scatter(wn_v, [pos // CHUNK, pos % CHUNK],
                         jnp.full((16,), 0, jnp.int32) + wn0, mask=pm)
      return 0
    lax.fori_loop(0, CHUNK // 16, pad_body, 0)

    copy_wait()

    def chunk_body(c, _):
      pltpu.make_async_copy(upd_hbm.at[wb_v.at[c]], rows_v, sem).start()
      pltpu.make_async_copy(upd_hbm.at[wb_v.at[c]], rows_v, sem).wait()
      pltpu.make_async_copy(rows_v, out_hbm.at[wn_v.at[c]], sem).start()
      pltpu.make_async_copy(rows_v, out_hbm.at[wn_v.at[c]], sem).wait()
      return 0
    lax.fori_loop(0, n_chunks, chunk_body, 0)

  @pl.when(m_w == 0)
  def _no_scatter():
    copy_wait()


def _sc_stage(mem, node_ids, updated):
  mesh = plsc.VectorSubcoreMesh(
      core_axis_name="c", subcore_axis_name="s", num_cores=NC, num_subcores=NS)
  kern = pl.kernel(
      _sc_body,
      out_type=jax.ShapeDtypeStruct((N_NODES, D), jnp.float32),
      mesh=mesh,
      compiler_params=pltpu.CompilerParams(
          needs_layout_passes=False, use_tc_tiling_on_sc=False),
      scratch_types=[
          pltpu.VMEM((B,), jnp.int32),
          pltpu.VMEM((TPAD,), jnp.int32),
          pltpu.VMEM((NCHUNKS, CHUNK), jnp.int32),
          pltpu.VMEM((NCHUNKS, CHUNK), jnp.int32),
          pltpu.VMEM((CHUNK, D), jnp.float32),
          pltpu.SemaphoreType.DMA,
          pltpu.SemaphoreType.DMA,
      ],
  )
  return kern(mem, node_ids, updated)


def kernel(mem, embeddings, times, map_centers, node_ids):
  updated = _tc_stage(embeddings, map_centers, times)
  return _sc_stage(mem, node_ids, updated)


# speedup vs baseline: 3.5251x; 3.5251x over previous
import functools

import jax
import jax.numpy as jnp
from jax import lax
from jax.experimental import pallas as pl
from jax.experimental.pallas import tpu as pltpu
from jax.experimental.pallas import tpu_sc as plsc

N_NODES = 100000
D = 64
N_CELLS = 1024
B = 16384


TB = 512
N_TILES = B // TB


def _tc_body(emb_ref, cent_ref, times_ref, upd_ref):
  emb = emb_ref[...]
  cent = cent_ref[...]
  e2 = jnp.sum(emb * emb, axis=1, keepdims=True)
  c2 = jnp.sum(cent * cent, axis=1)[None, :]
  dot = jax.lax.dot_general(
      emb, cent, (((1,), (1,)), ((), ())),
      preferred_element_type=jnp.float32)
  d2 = e2 - 2.0 * dot + c2
  minval = jnp.min(d2, axis=1, keepdims=True)
  iota = lax.broadcasted_iota(jnp.int32, (TB, N_CELLS), 1)
  idx = jnp.min(jnp.where(d2 == minval, iota, N_CELLS), axis=1)
  onehot = (iota == idx[:, None]).astype(jnp.float32)
  gathered = jax.lax.dot_general(
      onehot, cent, (((1,), (0,)), ((), ())),
      preferred_element_type=jnp.float32,
      precision=jax.lax.Precision.HIGHEST)
  t = times_ref[0, 0, :]
  w = jax.nn.sigmoid(t)[:, None]
  upd_ref[...] = w * emb + (1.0 - w) * gathered


def _tc_stage(embeddings, map_centers, times):
  times3 = times.reshape(N_TILES, 1, TB)
  return pl.pallas_call(
      _tc_body,
      grid=(N_TILES,),
      in_specs=[
          pl.BlockSpec((TB, D), lambda i: (i, 0)),
          pl.BlockSpec((N_CELLS, D), lambda i: (0, 0)),
          pl.BlockSpec((1, 1, TB), lambda i: (i, 0, 0)),
      ],
      out_specs=pl.BlockSpec((TB, D), lambda i: (i, 0)),
      out_shape=jax.ShapeDtypeStruct((B, D), jnp.float32),
  )(embeddings, map_centers, times3)


NC = 2
NS = 16
NW = NC * NS
RANGE = 3128
LAST = N_NODES - (NW - 1) * RANGE
TPAD = ((RANGE + 15) // 16) * 16
CHUNK = 128
CPR = 512
NCHUNKS = (RANGE + CHUNK - 1) // CHUNK
IDV = B // 16


def _sc_body(mem_hbm, ids_hbm, upd_hbm, out_hbm,
             ids_v, table_v, wb_v, wn_v, rows_v, cbuf_v, sem, lsem, ssem):
  wid = lax.axis_index("s") * NC + lax.axis_index("c")
  base = pl.multiple_of(wid * RANGE, 8)
  is_last = wid == NW - 1

  def slab_copy(nrows):
    sizes = [CPR] * (nrows // CPR)
    if nrows % CPR:
      sizes.append(nrows % CPR)
    loads, stores = [], []
    off = base
    for c, sz in enumerate(sizes):
      b = c % 2
      loads.append(pltpu.make_async_copy(
          mem_hbm.at[pl.ds(off, sz)], cbuf_v.at[pl.ds(b * CPR, sz)],
          lsem.at[b]))
      stores.append(pltpu.make_async_copy(
          cbuf_v.at[pl.ds(b * CPR, sz)], out_hbm.at[pl.ds(off, sz)],
          ssem.at[b]))
      off = off + sz
    n = len(sizes)
    for c in range(n):
      if c >= 2:
        stores[c - 2].wait()
      loads[c].start()
      if c >= 1:
        loads[c - 1].wait()
        stores[c - 1].start()
    loads[n - 1].wait()
    stores[n - 1].start()
    for c in range(max(n - 2, 0), n):
      stores[c].wait()

  def copy_all():
    @pl.when(jnp.logical_not(is_last))
    def _():
      slab_copy(RANGE)

    @pl.when(is_last)
    def _():
      slab_copy(LAST)

  pltpu.sync_copy(ids_hbm, ids_v)

  lanes = lax.iota(jnp.int32, 16)

  def init_body(k, _):
    table_v[pl.ds(k * 16, 16)] = jnp.full((16,), -1, jnp.int32)
    return 0
  lax.fori_loop(0, TPAD // 16, init_body, 0)

  def scan_body(i, _):
    ids16 = ids_v[pl.ds(i * 16, 16)]
    local = ids16 - base
    msk = (local >= 0) & (local < RANGE)
    ivec = lanes + i * 16
    plsc.store_scatter(table_v, [local], ivec, mask=msk)
    return 0
  lax.fori_loop(0, IDV, scan_body, 0)
  plsc.subcore_barrier()

  def compact_body(k, cnt):
    ent = table_v[pl.ds(k * 16, 16)]
    m = ent >= 0
    pos = cnt + plsc.cumsum(m.astype(jnp.int32)) - 1
    node = lanes + (k * 16 + base)
    plsc.store_scatter(wb_v, [pos // CHUNK, pos % CHUNK], ent, mask=m)
    plsc.store_scatter(wn_v, [pos // CHUNK, pos % CHUNK], node, mask=m)
    return cnt + jnp.sum(m.astype(jnp.int32))
  m_w = lax.fori_loop(0, TPAD // 16, compact_body, 0)
  plsc.subcore_barrier()

  n_chunks = (m_w + CHUNK - 1) // CHUNK

  @pl.when(m_w > 0)
  def _do_pad():
    wb16 = wb_v[0, pl.ds(0, 16)]
    wn16 = wn_v[0, pl.ds(0, 16)]
    sel0 = (lanes == 0).astype(jnp.int32)
    wb0 = jnp.sum(wb16 * sel0)
    wn0 = jnp.sum(wn16 * sel0)
    padded = n_chunks * CHUNK

    def pad_body(j, _):
      pos = m_w + j * 16 + lanes
      pm = pos < padded
      plsc.store_scatter(wb_v, [pos // CHUNK, pos % CHUNK],
                         jnp.full((16,), 0, jnp.int32) + wb0, mask=pm)
      plsc.store_scatter(wn_v, [pos // CHUNK, pos % CHUNK],
                         jnp.full((16,), 0, jnp.int32) + wn0, mask=pm)
      return 0
    lax.fori_loop(0, CHUNK // 16, pad_body, 0)

  copy_all()

  @pl.when(m_w > 0)
  def _do_scatter():
    def chunk_body(c, _):
      pltpu.make_async_copy(upd_hbm.at[wb_v.at[c]], rows_v, sem).start()
      pltpu.make_async_copy(upd_hbm.at[wb_v.at[c]], rows_v, sem).wait()
      pltpu.make_async_copy(rows_v, out_hbm.at[wn_v.at[c]], sem).start()
      pltpu.make_async_copy(rows_v, out_hbm.at[wn_v.at[c]], sem).wait()
      return 0
    lax.fori_loop(0, n_chunks, chunk_body, 0)


def _sc_stage(mem, node_ids, updated):
  mesh = plsc.VectorSubcoreMesh(
      core_axis_name="c", subcore_axis_name="s", num_cores=NC, num_subcores=NS)
  kern = pl.kernel(
      _sc_body,
      out_type=jax.ShapeDtypeStruct((N_NODES, D), jnp.float32),
      mesh=mesh,
      compiler_params=pltpu.CompilerParams(
          needs_layout_passes=False, use_tc_tiling_on_sc=False),
      scratch_types=[
          pltpu.VMEM((B,), jnp.int32),
          pltpu.VMEM((TPAD,), jnp.int32),
          pltpu.VMEM((NCHUNKS, CHUNK), jnp.int32),
          pltpu.VMEM((NCHUNKS, CHUNK), jnp.int32),
          pltpu.VMEM((CHUNK, D), jnp.float32),
          pltpu.VMEM((2 * CPR, D), jnp.float32),
          pltpu.SemaphoreType.DMA,
          pltpu.SemaphoreType.DMA((2,)),
          pltpu.SemaphoreType.DMA((2,)),
      ],
  )
  return kern(mem, node_ids, updated)


def kernel(mem, embeddings, times, map_centers, node_ids):
  updated = _tc_stage(embeddings, map_centers, times)
  return _sc_stage(mem, node_ids, updated)


# speedup vs baseline: 4.1616x; 1.1805x over previous
import functools

import jax
import jax.numpy as jnp
from jax import lax
from jax.experimental import pallas as pl
from jax.experimental.pallas import tpu as pltpu
from jax.experimental.pallas import tpu_sc as plsc

N_NODES = 100000
D = 64
N_CELLS = 1024
B = 16384


TB = 512
N_TILES = B // TB


def _tc_body(emb_ref, cent_ref, times_ref, upd_ref):
  emb = emb_ref[...]
  cent = cent_ref[...]
  e2 = jnp.sum(emb * emb, axis=1, keepdims=True)
  c2 = jnp.sum(cent * cent, axis=1)[None, :]
  dot = jax.lax.dot_general(
      emb, cent, (((1,), (1,)), ((), ())),
      preferred_element_type=jnp.float32)
  d2 = e2 - 2.0 * dot + c2
  minval = jnp.min(d2, axis=1, keepdims=True)
  iota = lax.broadcasted_iota(jnp.int32, (TB, N_CELLS), 1)
  idx = jnp.min(jnp.where(d2 == minval, iota, N_CELLS), axis=1)
  onehot = (iota == idx[:, None]).astype(jnp.float32)
  gathered = jax.lax.dot_general(
      onehot, cent, (((1,), (0,)), ((), ())),
      preferred_element_type=jnp.float32)
  t = times_ref[0, 0, :]
  w = jax.nn.sigmoid(t)[:, None]
  upd_ref[...] = w * emb + (1.0 - w) * gathered


def _tc_stage(embeddings, map_centers, times):
  times3 = times.reshape(N_TILES, 1, TB)
  return pl.pallas_call(
      _tc_body,
      grid=(N_TILES,),
      in_specs=[
          pl.BlockSpec((TB, D), lambda i: (i, 0)),
          pl.BlockSpec((N_CELLS, D), lambda i: (0, 0)),
          pl.BlockSpec((1, 1, TB), lambda i: (i, 0, 0)),
      ],
      out_specs=pl.BlockSpec((TB, D), lambda i: (i, 0)),
      out_shape=jax.ShapeDtypeStruct((B, D), jnp.float32),
  )(embeddings, map_centers, times3)


NC = 2
NS = 16
NW = NC * NS
RANGE = 3128
LAST = N_NODES - (NW - 1) * RANGE
TPAD = ((RANGE + 15) // 16) * 16
CHUNK = 128
CPR = 512
NCHUNKS = (RANGE + CHUNK - 1) // CHUNK
IDV = B // 16


def _sc_body(mem_hbm, ids_hbm, upd_hbm, out_hbm,
             ids_v, table_v, wb_v, wn_v, rows_v, cbuf_v, sem, lsem, ssem):
  wid = lax.axis_index("s") * NC + lax.axis_index("c")
  base = pl.multiple_of(wid * RANGE, 8)
  is_last = wid == NW - 1

  def slab_copy(nrows):
    sizes = [CPR] * (nrows // CPR)
    if nrows % CPR:
      sizes.append(nrows % CPR)
    loads, stores = [], []
    off = base
    for c, sz in enumerate(sizes):
      b = c % 2
      loads.append(pltpu.make_async_copy(
          mem_hbm.at[pl.ds(off, sz)], cbuf_v.at[pl.ds(b * CPR, sz)],
          lsem.at[b]))
      stores.append(pltpu.make_async_copy(
          cbuf_v.at[pl.ds(b * CPR, sz)], out_hbm.at[pl.ds(off, sz)],
          ssem.at[b]))
      off = off + sz
    n = len(sizes)
    for c in range(n):
      if c >= 2:
        stores[c - 2].wait()
      loads[c].start()
      if c >= 1:
        loads[c - 1].wait()
        stores[c - 1].start()
    loads[n - 1].wait()
    stores[n - 1].start()
    for c in range(max(n - 2, 0), n):
      stores[c].wait()

  def copy_all():
    @pl.when(jnp.logical_not(is_last))
    def _():
      slab_copy(RANGE)

    @pl.when(is_last)
    def _():
      slab_copy(LAST)

  pltpu.sync_copy(ids_hbm, ids_v)

  lanes = lax.iota(jnp.int32, 16)

  def init_body(k, _):
    table_v[pl.ds(k * 16, 16)] = jnp.full((16,), -1, jnp.int32)
    return 0
  lax.fori_loop(0, TPAD // 16, init_body, 0)

  def scan_body(i, _):
    ids16 = ids_v[pl.ds(i * 16, 16)]
    local = ids16 - base
    msk = (local >= 0) & (local < RANGE)
    ivec = lanes + i * 16
    plsc.store_scatter(table_v, [local], ivec, mask=msk)
    return 0
  lax.fori_loop(0, IDV, scan_body, 0, unroll=4)
  plsc.subcore_barrier()

  def compact_body(k, cnt):
    ent = table_v[pl.ds(k * 16, 16)]
    m = ent >= 0
    pos = cnt + plsc.cumsum(m.astype(jnp.int32)) - 1
    node = lanes + (k * 16 + base)
    plsc.store_scatter(wb_v, [pos // CHUNK, pos % CHUNK], ent, mask=m)
    plsc.store_scatter(wn_v, [pos // CHUNK, pos % CHUNK], node, mask=m)
    return cnt + jnp.sum(m.astype(jnp.int32))
  m_w = lax.fori_loop(0, TPAD // 16, compact_body, 0)
  plsc.subcore_barrier()

  n_chunks = (m_w + CHUNK - 1) // CHUNK

  @pl.when(m_w > 0)
  def _do_pad():
    wb16 = wb_v[0, pl.ds(0, 16)]
    wn16 = wn_v[0, pl.ds(0, 16)]
    sel0 = (lanes == 0).astype(jnp.int32)
    wb0 = jnp.sum(wb16 * sel0)
    wn0 = jnp.sum(wn16 * sel0)
    padded = n_chunks * CHUNK

    def pad_body(j, _):
      pos = m_w + j * 16 + lanes
      pm = pos < padded
      plsc.store_scatter(wb_v, [pos // CHUNK, pos % CHUNK],
                         jnp.full((16,), 0, jnp.int32) + wb0, mask=pm)
      plsc.store_scatter(wn_v, [pos // CHUNK, pos % CHUNK],
                         jnp.full((16,), 0, jnp.int32) + wn0, mask=pm)
      return 0
    lax.fori_loop(0, CHUNK // 16, pad_body, 0)

  copy_all()

  @pl.when(m_w > 0)
  def _do_scatter():
    def chunk_body(c, _):
      pltpu.make_async_copy(upd_hbm.at[wb_v.at[c]], rows_v, sem).start()
      pltpu.make_async_copy(upd_hbm.at[wb_v.at[c]], rows_v, sem).wait()
      pltpu.make_async_copy(rows_v, out_hbm.at[wn_v.at[c]], sem).start()
      pltpu.make_async_copy(rows_v, out_hbm.at[wn_v.at[c]], sem).wait()
      return 0
    lax.fori_loop(0, n_chunks, chunk_body, 0)


def _sc_stage(mem, node_ids, updated):
  mesh = plsc.VectorSubcoreMesh(
      core_axis_name="c", subcore_axis_name="s", num_cores=NC, num_subcores=NS)
  kern = pl.kernel(
      _sc_body,
      out_type=jax.ShapeDtypeStruct((N_NODES, D), jnp.float32),
      mesh=mesh,
      compiler_params=pltpu.CompilerParams(
          needs_layout_passes=False, use_tc_tiling_on_sc=False),
      scratch_types=[
          pltpu.VMEM((B,), jnp.int32),
          pltpu.VMEM((TPAD,), jnp.int32),
          pltpu.VMEM((NCHUNKS, CHUNK), jnp.int32),
          pltpu.VMEM((NCHUNKS, CHUNK), jnp.int32),
          pltpu.VMEM((CHUNK, D), jnp.float32),
          pltpu.VMEM((2 * CPR, D), jnp.float32),
          pltpu.SemaphoreType.DMA,
          pltpu.SemaphoreType.DMA((2,)),
          pltpu.SemaphoreType.DMA((2,)),
      ],
  )
  return kern(mem, node_ids, updated)


def kernel(mem, embeddings, times, map_centers, node_ids):
  updated = _tc_stage(embeddings, map_centers, times)
  return _sc_stage(mem, node_ids, updated)


# speedup vs baseline: 5.2565x; 1.2631x over previous
import jax
import jax.numpy as jnp
from jax import lax
from jax.experimental import pallas as pl
from jax.experimental.pallas import tpu as pltpu
from jax.experimental.pallas import tpu_sc as plsc
from jax._src.pallas import mpmd as pl_mpmd

N_NODES = 100000
D = 64
DP = 128
N_CELLS = 1024
B = 16384


TB = 512
N_TILES = B // TB
MEMB = 3128


def _tc_body(emb_ref, cent_ref, times_ref, mem_ref, upd_ref, memc_ref):
  emb = emb_ref[...]
  cent = cent_ref[...]
  e2 = jnp.sum(emb * emb, axis=1, keepdims=True)
  c2 = jnp.sum(cent * cent, axis=1)[None, :]
  dot = jax.lax.dot_general(
      emb, cent, (((1,), (1,)), ((), ())),
      preferred_element_type=jnp.float32)
  d2 = e2 - 2.0 * dot + c2
  minval = jnp.min(d2, axis=1, keepdims=True)
  iota = lax.broadcasted_iota(jnp.int32, (TB, N_CELLS), 1)
  idx = jnp.min(jnp.where(d2 == minval, iota, N_CELLS), axis=1)
  onehot = (iota == idx[:, None]).astype(jnp.float32)
  gathered = jax.lax.dot_general(
      onehot, cent, (((1,), (0,)), ((), ())),
      preferred_element_type=jnp.float32)
  t = times_ref[0, 0, :]
  w = jax.nn.sigmoid(t)[:, None]
  upd = w * emb + (1.0 - w) * gathered
  zpad = jnp.zeros((TB, DP - D), jnp.float32)
  upd_ref[...] = jnp.concatenate([upd, zpad], axis=1)
  memc_ref[...] = jnp.concatenate(
      [mem_ref[...], jnp.zeros((MEMB, DP - D), jnp.float32)], axis=1)


def _tc_stage(embeddings, map_centers, times, mem):
  times3 = times.reshape(N_TILES, 1, TB)
  return pl.pallas_call(
      _tc_body,
      grid=(N_TILES,),
      in_specs=[
          pl.BlockSpec((TB, D), lambda i: (i, 0)),
          pl.BlockSpec((N_CELLS, D), lambda i: (0, 0)),
          pl.BlockSpec((1, 1, TB), lambda i: (i, 0, 0)),
          pl.BlockSpec((MEMB, D), lambda i: (i, 0)),
      ],
      out_specs=[
          pl.BlockSpec((TB, DP), lambda i: (i, 0)),
          pl.BlockSpec((MEMB, DP), lambda i: (i, 0)),
      ],
      out_shape=[
          jax.ShapeDtypeStruct((B, DP), jnp.float32),
          jax.ShapeDtypeStruct((N_NODES, DP), jnp.float32),
      ],
  )(embeddings, map_centers, times3, mem)


NC = 2
NS = 16
NW = NC * NS
RANGE = 3128
TPAD = ((RANGE + 15) // 16) * 16
CHUNK = 128
NCHUNKS = (RANGE + CHUNK - 1) // CHUNK
IDV = B // 16


def _sc_body(mem_hbm, ids_hbm, upd_hbm, out_hbm,
             ids_v, table_v, wb_v, wn_v, rows_v, sem):
  wid = lax.axis_index("s") * NC + lax.axis_index("c")
  base = wid * RANGE

  pltpu.sync_copy(ids_hbm, ids_v)

  lanes = lax.iota(jnp.int32, 16)

  def init_body(k, _):
    table_v[pl.ds(k * 16, 16)] = jnp.full((16,), -1, jnp.int32)
    return 0
  lax.fori_loop(0, TPAD // 16, init_body, 0)

  def scan_body(i, _):
    ids16 = ids_v[pl.ds(i * 16, 16)]
    local = ids16 - base
    msk = (local >= 0) & (local < RANGE)
    ivec = lanes + i * 16
    plsc.store_scatter(table_v, [local], ivec, mask=msk)
    return 0
  lax.fori_loop(0, IDV, scan_body, 0, unroll=4)
  plsc.subcore_barrier()

  def compact_body(k, cnt):
    ent = table_v[pl.ds(k * 16, 16)]
    m = ent >= 0
    pos = cnt + plsc.cumsum(m.astype(jnp.int32)) - 1
    node = lanes + (k * 16 + base)
    plsc.store_scatter(wb_v, [pos // CHUNK, pos % CHUNK], ent, mask=m)
    plsc.store_scatter(wn_v, [pos // CHUNK, pos % CHUNK], node, mask=m)
    return cnt + jnp.sum(m.astype(jnp.int32))
  m_w = lax.fori_loop(0, TPAD // 16, compact_body, 0)
  plsc.subcore_barrier()

  n_chunks = (m_w + CHUNK - 1) // CHUNK

  @pl.when(m_w > 0)
  def _do_pad():
    wb16 = wb_v[0, pl.ds(0, 16)]
    wn16 = wn_v[0, pl.ds(0, 16)]
    sel0 = (lanes == 0).astype(jnp.int32)
    wb0 = jnp.sum(wb16 * sel0)
    wn0 = jnp.sum(wn16 * sel0)
    padded = n_chunks * CHUNK

    def pad_body(j, _):
      pos = m_w + j * 16 + lanes
      pm = pos < padded
      plsc.store_scatter(wb_v, [pos // CHUNK, pos % CHUNK],
                         jnp.full((16,), 0, jnp.int32) + wb0, mask=pm)
      plsc.store_scatter(wn_v, [pos // CHUNK, pos % CHUNK],
                         jnp.full((16,), 0, jnp.int32) + wn0, mask=pm)
      return 0
    lax.fori_loop(0, CHUNK // 16, pad_body, 0)

  plsc.subcore_barrier()

  @pl.when(m_w > 0)
  def _do_scatter():
    def chunk_body(c, _):
      pltpu.make_async_copy(upd_hbm.at[wb_v.at[c]], rows_v, sem).start()
      pltpu.make_async_copy(upd_hbm.at[wb_v.at[c]], rows_v, sem).wait()
      pltpu.make_async_copy(rows_v, out_hbm.at[wn_v.at[c]], sem).start()
      pltpu.make_async_copy(rows_v, out_hbm.at[wn_v.at[c]], sem).wait()
      return 0
    lax.fori_loop(0, n_chunks, chunk_body, 0)


def _sc_stage(mem128, node_ids, updated):
  mesh = plsc.VectorSubcoreMesh(
      core_axis_name="c", subcore_axis_name="s", num_cores=NC, num_subcores=NS)
  kern = pl_mpmd._mpmd_map(
      [(mesh, _sc_body)],
      [jax.ShapeDtypeStruct((N_NODES, DP), jnp.float32)],
      input_output_aliases={0: 0},
      compiler_params=pltpu.CompilerParams(
          needs_layout_passes=False, use_tc_tiling_on_sc=True),
      scratch_types=[
          pltpu.VMEM((B,), jnp.int32),
          pltpu.VMEM((TPAD,), jnp.int32),
          pltpu.VMEM((NCHUNKS, CHUNK), jnp.int32),
          pltpu.VMEM((NCHUNKS, CHUNK), jnp.int32),
          pltpu.VMEM((CHUNK, DP), jnp.float32),
          pltpu.SemaphoreType.DMA,
      ],
  )
  return kern(mem128, node_ids, updated)[0]


def kernel(mem, embeddings, times, map_centers, node_ids):
  updated, mem128 = _tc_stage(embeddings, map_centers, times, mem)
  out128 = _sc_stage(mem128, node_ids, updated)
  return out128[:, :D]


# speedup vs baseline: 6.8162x; 1.2967x over previous
import jax
import jax.numpy as jnp
from jax import lax
from jax.experimental import pallas as pl
from jax.experimental.pallas import tpu as pltpu
from jax.experimental.pallas import tpu_sc as plsc
from jax._src.pallas import mpmd as pl_mpmd

N_NODES = 100000
D = 64
DP = 128
N_CELLS = 1024
B = 16384


TB = 512
N_TILES = B // TB
MEMB = 3200


def _tc_body(emb_ref, cent_ref, times_ref, mem_ref, upd_ref, memc_ref):
  emb = emb_ref[...]
  cent = cent_ref[...]
  e2 = jnp.sum(emb * emb, axis=1, keepdims=True)
  c2 = jnp.sum(cent * cent, axis=1)[None, :]
  dot = jax.lax.dot_general(
      emb, cent, (((1,), (1,)), ((), ())),
      preferred_element_type=jnp.float32)
  d2 = e2 - 2.0 * dot + c2
  minval = jnp.min(d2, axis=1, keepdims=True)
  iota = lax.broadcasted_iota(jnp.int32, (TB, N_CELLS), 1)
  idx = jnp.min(jnp.where(d2 == minval, iota, N_CELLS), axis=1)
  onehot = (iota == idx[:, None]).astype(jnp.float32)
  gathered = jax.lax.dot_general(
      onehot, cent, (((1,), (0,)), ((), ())),
      preferred_element_type=jnp.float32)
  t = times_ref[0, 0, :]
  w = jax.nn.sigmoid(t)[:, None]
  upd = w * emb + (1.0 - w) * gathered
  zpad = jnp.zeros((TB, DP - D), jnp.float32)
  upd_ref[...] = jnp.concatenate([upd, zpad], axis=1)
  memc_ref[...] = jnp.concatenate(
      [mem_ref[...].T, jnp.zeros((MEMB, DP - D), jnp.float32)], axis=1)


def _tc_stage(embeddings, map_centers, times, mem):
  times3 = times.reshape(N_TILES, 1, TB)
  return pl.pallas_call(
      _tc_body,
      grid=(N_TILES,),
      in_specs=[
          pl.BlockSpec((TB, D), lambda i: (i, 0)),
          pl.BlockSpec((N_CELLS, D), lambda i: (0, 0)),
          pl.BlockSpec((1, 1, TB), lambda i: (i, 0, 0)),
          pl.BlockSpec((D, MEMB), lambda i: (0, i)),
      ],
      out_specs=[
          pl.BlockSpec((TB, DP), lambda i: (i, 0)),
          pl.BlockSpec((MEMB, DP), lambda i: (i, 0)),
      ],
      out_shape=[
          jax.ShapeDtypeStruct((B, DP), jnp.float32),
          jax.ShapeDtypeStruct((N_NODES, DP), jnp.float32),
      ],
  )(embeddings, map_centers, times3, mem.T)


NC = 2
NS = 16
NW = NC * NS
RANGE = 3128
TPAD = ((RANGE + 15) // 16) * 16
CHUNK = 128
NCHUNKS = (RANGE + CHUNK - 1) // CHUNK
IDV = B // 16


def _sc_body(mem_hbm, ids_hbm, upd_hbm, out_hbm,
             ids_v, table_v, wb_v, wn_v, rows_v, sem):
  wid = lax.axis_index("s") * NC + lax.axis_index("c")
  base = wid * RANGE

  pltpu.sync_copy(ids_hbm, ids_v)

  lanes = lax.iota(jnp.int32, 16)

  def init_body(k, _):
    table_v[pl.ds(k * 16, 16)] = jnp.full((16,), -1, jnp.int32)
    return 0
  lax.fori_loop(0, TPAD // 16, init_body, 0)

  def scan_body(i, _):
    ids16 = ids_v[pl.ds(i * 16, 16)]
    local = ids16 - base
    msk = (local >= 0) & (local < RANGE)
    ivec = lanes + i * 16
    plsc.store_scatter(table_v, [local], ivec, mask=msk)
    return 0
  lax.fori_loop(0, IDV, scan_body, 0, unroll=4)
  plsc.subcore_barrier()

  def compact_body(k, cnt):
    ent = table_v[pl.ds(k * 16, 16)]
    m = ent >= 0
    pos = cnt + plsc.cumsum(m.astype(jnp.int32)) - 1
    node = lanes + (k * 16 + base)
    plsc.store_scatter(wb_v, [pos // CHUNK, pos % CHUNK], ent, mask=m)
    plsc.store_scatter(wn_v, [pos // CHUNK, pos % CHUNK], node, mask=m)
    return cnt + jnp.sum(m.astype(jnp.int32))
  m_w = lax.fori_loop(0, TPAD // 16, compact_body, 0)
  plsc.subcore_barrier()

  n_chunks = (m_w + CHUNK - 1) // CHUNK

  @pl.when(m_w > 0)
  def _do_pad():
    wb16 = wb_v[0, pl.ds(0, 16)]
    wn16 = wn_v[0, pl.ds(0, 16)]
    sel0 = (lanes == 0).astype(jnp.int32)
    wb0 = jnp.sum(wb16 * sel0)
    wn0 = jnp.sum(wn16 * sel0)
    padded = n_chunks * CHUNK

    def pad_body(j, _):
      pos = m_w + j * 16 + lanes
      pm = pos < padded
      plsc.store_scatter(wb_v, [pos // CHUNK, pos % CHUNK],
                         jnp.full((16,), 0, jnp.int32) + wb0, mask=pm)
      plsc.store_scatter(wn_v, [pos // CHUNK, pos % CHUNK],
                         jnp.full((16,), 0, jnp.int32) + wn0, mask=pm)
      return 0
    lax.fori_loop(0, CHUNK // 16, pad_body, 0)

  plsc.subcore_barrier()

  @pl.when(m_w > 0)
  def _do_scatter():
    def chunk_body(c, _):
      pltpu.make_async_copy(upd_hbm.at[wb_v.at[c]], rows_v, sem).start()
      pltpu.make_async_copy(upd_hbm.at[wb_v.at[c]], rows_v, sem).wait()
      pltpu.make_async_copy(rows_v, out_hbm.at[wn_v.at[c]], sem).start()
      pltpu.make_async_copy(rows_v, out_hbm.at[wn_v.at[c]], sem).wait()
      return 0
    lax.fori_loop(0, n_chunks, chunk_body, 0)


def _sc_stage(mem128, node_ids, updated):
  mesh = plsc.VectorSubcoreMesh(
      core_axis_name="c", subcore_axis_name="s", num_cores=NC, num_subcores=NS)
  kern = pl_mpmd._mpmd_map(
      [(mesh, _sc_body)],
      [jax.ShapeDtypeStruct((N_NODES, DP), jnp.float32)],
      input_output_aliases={0: 0},
      compiler_params=pltpu.CompilerParams(
          needs_layout_passes=False, use_tc_tiling_on_sc=True),
      scratch_types=[
          pltpu.VMEM((B,), jnp.int32),
          pltpu.VMEM((TPAD,), jnp.int32),
          pltpu.VMEM((NCHUNKS, CHUNK), jnp.int32),
          pltpu.VMEM((NCHUNKS, CHUNK), jnp.int32),
          pltpu.VMEM((CHUNK, DP), jnp.float32),
          pltpu.SemaphoreType.DMA,
      ],
  )
  return kern(mem128, node_ids, updated)[0]


def kernel(mem, embeddings, times, map_centers, node_ids):
  updated, mem128 = _tc_stage(embeddings, map_centers, times, mem)
  out128 = _sc_stage(mem128, node_ids, updated)
  return out128[:, :D]


# speedup vs baseline: 7.1152x; 1.0439x over previous
import jax
import jax.numpy as jnp
from jax import lax
from jax.experimental import pallas as pl
from jax.experimental.pallas import tpu as pltpu
from jax.experimental.pallas import tpu_sc as plsc
from jax._src.pallas import mpmd as pl_mpmd

N_NODES = 100000
D = 64
DP = 128
N_CELLS = 1024
B = 16384


TB = 512
N_TILES = B // TB
MEMB = 3200


def _tc_body(emb_ref, cent_ref, times_ref, mem_ref, upd_ref, memc_ref):
  emb = emb_ref[...].T
  cent = cent_ref[...].T
  e2 = jnp.sum(emb * emb, axis=1, keepdims=True)
  c2 = jnp.sum(cent * cent, axis=1)[None, :]
  dot = jax.lax.dot_general(
      emb, cent, (((1,), (1,)), ((), ())),
      preferred_element_type=jnp.float32)
  d2 = e2 - 2.0 * dot + c2
  minval = jnp.min(d2, axis=1, keepdims=True)
  iota = lax.broadcasted_iota(jnp.int32, (TB, N_CELLS), 1)
  idx = jnp.min(jnp.where(d2 == minval, iota, N_CELLS), axis=1)
  onehot = (iota == idx[:, None]).astype(jnp.float32)
  gathered = jax.lax.dot_general(
      onehot, cent, (((1,), (0,)), ((), ())),
      preferred_element_type=jnp.float32)
  t = times_ref[0, 0, :]
  w = jax.nn.sigmoid(t)[:, None]
  upd = w * emb + (1.0 - w) * gathered
  zpad = jnp.zeros((TB, DP - D), jnp.float32)
  upd_ref[...] = jnp.concatenate([upd, zpad], axis=1)
  memc_ref[...] = jnp.concatenate(
      [mem_ref[...].T, jnp.zeros((MEMB, DP - D), jnp.float32)], axis=1)


def _tc_stage(embeddings, map_centers, times, mem):
  times3 = times.reshape(N_TILES, 1, TB)
  return pl.pallas_call(
      _tc_body,
      grid=(N_TILES,),
      in_specs=[
          pl.BlockSpec((D, TB), lambda i: (0, i)),
          pl.BlockSpec((D, N_CELLS), lambda i: (0, 0)),
          pl.BlockSpec((1, 1, TB), lambda i: (i, 0, 0)),
          pl.BlockSpec((D, MEMB), lambda i: (0, i)),
      ],
      out_specs=[
          pl.BlockSpec((TB, DP), lambda i: (i, 0)),
          pl.BlockSpec((MEMB, DP), lambda i: (i, 0)),
      ],
      out_shape=[
          jax.ShapeDtypeStruct((B, DP), jnp.float32),
          jax.ShapeDtypeStruct((N_NODES, DP), jnp.float32),
      ],
  )(embeddings.T, map_centers.T, times3, mem.T)


NC = 2
NS = 16
NW = NC * NS
RANGE = 3128
TPAD = ((RANGE + 15) // 16) * 16
CHUNK = 128
NCHUNKS = (RANGE + CHUNK - 1) // CHUNK
IDV = B // 16


def _sc_body(mem_hbm, ids_hbm, upd_hbm, out_hbm,
             ids_v, table_v, wb_v, wn_v, rows_v, sem):
  wid = lax.axis_index("s") * NC + lax.axis_index("c")
  base = wid * RANGE

  pltpu.sync_copy(ids_hbm, ids_v)

  lanes = lax.iota(jnp.int32, 16)

  def init_body(k, _):
    table_v[pl.ds(k * 16, 16)] = jnp.full((16,), -1, jnp.int32)
    return 0
  lax.fori_loop(0, TPAD // 16, init_body, 0)

  def scan_body(i, _):
    ids16 = ids_v[pl.ds(i * 16, 16)]
    local = ids16 - base
    msk = (local >= 0) & (local < RANGE)
    ivec = lanes + i * 16
    plsc.store_scatter(table_v, [local], ivec, mask=msk)
    return 0
  lax.fori_loop(0, IDV, scan_body, 0, unroll=4)
  plsc.subcore_barrier()

  def compact_body(k, cnt):
    ent = table_v[pl.ds(k * 16, 16)]
    m = ent >= 0
    pos = cnt + plsc.cumsum(m.astype(jnp.int32)) - 1
    node = lanes + (k * 16 + base)
    plsc.store_scatter(wb_v, [pos // CHUNK, pos % CHUNK], ent, mask=m)
    plsc.store_scatter(wn_v, [pos // CHUNK, pos % CHUNK], node, mask=m)
    return cnt + jnp.sum(m.astype(jnp.int32))
  m_w = lax.fori_loop(0, TPAD // 16, compact_body, 0)
  plsc.subcore_barrier()

  n_chunks = (m_w + CHUNK - 1) // CHUNK

  @pl.when(m_w > 0)
  def _do_pad():
    wb16 = wb_v[0, pl.ds(0, 16)]
    wn16 = wn_v[0, pl.ds(0, 16)]
    sel0 = (lanes == 0).astype(jnp.int32)
    wb0 = jnp.sum(wb16 * sel0)
    wn0 = jnp.sum(wn16 * sel0)
    padded = n_chunks * CHUNK

    def pad_body(j, _):
      pos = m_w + j * 16 + lanes
      pm = pos < padded
      plsc.store_scatter(wb_v, [pos // CHUNK, pos % CHUNK],
                         jnp.full((16,), 0, jnp.int32) + wb0, mask=pm)
      plsc.store_scatter(wn_v, [pos // CHUNK, pos % CHUNK],
                         jnp.full((16,), 0, jnp.int32) + wn0, mask=pm)
      return 0
    lax.fori_loop(0, CHUNK // 16, pad_body, 0)

  plsc.subcore_barrier()

  @pl.when(m_w > 0)
  def _do_scatter():
    def chunk_body(c, _):
      pltpu.make_async_copy(upd_hbm.at[wb_v.at[c]], rows_v, sem).start()
      pltpu.make_async_copy(upd_hbm.at[wb_v.at[c]], rows_v, sem).wait()
      pltpu.make_async_copy(rows_v, out_hbm.at[wn_v.at[c]], sem).start()
      pltpu.make_async_copy(rows_v, out_hbm.at[wn_v.at[c]], sem).wait()
      return 0
    lax.fori_loop(0, n_chunks, chunk_body, 0)


def _sc_stage(mem128, node_ids, updated):
  mesh = plsc.VectorSubcoreMesh(
      core_axis_name="c", subcore_axis_name="s", num_cores=NC, num_subcores=NS)
  kern = pl_mpmd._mpmd_map(
      [(mesh, _sc_body)],
      [jax.ShapeDtypeStruct((N_NODES, DP), jnp.float32)],
      input_output_aliases={0: 0},
      compiler_params=pltpu.CompilerParams(
          needs_layout_passes=False, use_tc_tiling_on_sc=True),
      scratch_types=[
          pltpu.VMEM((B,), jnp.int32),
          pltpu.VMEM((TPAD,), jnp.int32),
          pltpu.VMEM((NCHUNKS, CHUNK), jnp.int32),
          pltpu.VMEM((NCHUNKS, CHUNK), jnp.int32),
          pltpu.VMEM((CHUNK, DP), jnp.float32),
          pltpu.SemaphoreType.DMA,
      ],
  )
  return kern(mem128, node_ids, updated)[0]


def kernel(mem, embeddings, times, map_centers, node_ids):
  updated, mem128 = _tc_stage(embeddings, map_centers, times, mem)
  out128 = _sc_stage(mem128, node_ids, updated)
  return out128[:, :D]


# speedup vs baseline: 7.1912x; 1.0107x over previous
import jax
import jax.numpy as jnp
from jax import lax
from jax.experimental import pallas as pl
from jax.experimental.pallas import tpu as pltpu
from jax.experimental.pallas import tpu_sc as plsc
from jax._src.pallas import mpmd as pl_mpmd

N_NODES = 100000
D = 64
DP = 128
N_CELLS = 1024
B = 16384


TB = 512
N_TILES = B // TB
MEMB = 3200


def _tc_body(emb_ref, cent_ref, times_ref, mem_ref, upd_ref, memc_ref):
  emb = emb_ref[...].T
  cent = cent_ref[...].T
  e2 = jnp.sum(emb * emb, axis=1, keepdims=True)
  c2 = jnp.sum(cent * cent, axis=1)[None, :]
  dot = jax.lax.dot_general(
      emb, cent, (((1,), (1,)), ((), ())),
      preferred_element_type=jnp.float32)
  d2 = e2 - 2.0 * dot + c2
  minval = jnp.min(d2, axis=1, keepdims=True)
  iota = lax.broadcasted_iota(jnp.int32, (TB, N_CELLS), 1)
  idx = jnp.min(jnp.where(d2 == minval, iota, N_CELLS), axis=1)
  onehot = (iota == idx[:, None]).astype(jnp.float32)
  gathered = jax.lax.dot_general(
      onehot, cent, (((1,), (0,)), ((), ())),
      preferred_element_type=jnp.float32)
  t = times_ref[0, 0, :]
  w = jax.nn.sigmoid(t)[:, None]
  upd = w * emb + (1.0 - w) * gathered
  zpad = jnp.zeros((TB, DP - D), jnp.float32)
  upd_ref[...] = jnp.concatenate([upd, zpad], axis=1)
  memc_ref[...] = jnp.concatenate(
      [mem_ref[...].T, jnp.zeros((MEMB, DP - D), jnp.float32)], axis=1)


def _tc_stage(embeddings, map_centers, times, mem):
  times3 = times.reshape(N_TILES, 1, TB)
  return pl.pallas_call(
      _tc_body,
      grid=(N_TILES,),
      in_specs=[
          pl.BlockSpec((D, TB), lambda i: (0, i)),
          pl.BlockSpec((D, N_CELLS), lambda i: (0, 0)),
          pl.BlockSpec((1, 1, TB), lambda i: (i, 0, 0)),
          pl.BlockSpec((D, MEMB), lambda i: (0, i)),
      ],
      out_specs=[
          pl.BlockSpec((TB, DP), lambda i: (i, 0)),
          pl.BlockSpec((MEMB, DP), lambda i: (i, 0)),
      ],
      out_shape=[
          jax.ShapeDtypeStruct((B, DP), jnp.float32),
          jax.ShapeDtypeStruct((N_NODES, DP), jnp.float32),
      ],
  )(embeddings.T, map_centers.T, times3, mem.T)


NC = 2
NS = 16
NW = NC * NS
RANGE = 3128
TPAD = ((RANGE + 15) // 16) * 16
CHUNK = 128
NCHUNKS = (RANGE + CHUNK - 1) // CHUNK
IDV = B // 16


def _sc_body(mem_hbm, ids_hbm, upd_hbm, out_hbm,
             ids_v, table_v, wb_v, wn_v, rows_v, sem, gsem, ssem):
  wid = lax.axis_index("s") * NC + lax.axis_index("c")
  base = wid * RANGE

  pltpu.sync_copy(ids_hbm, ids_v)

  lanes = lax.iota(jnp.int32, 16)

  def init_body(k, _):
    table_v[pl.ds(k * 16, 16)] = jnp.full((16,), -1, jnp.int32)
    return 0
  lax.fori_loop(0, TPAD // 16, init_body, 0)

  def scan_body(i, _):
    ids16 = ids_v[pl.ds(i * 16, 16)]
    local = ids16 - base
    msk = (local >= 0) & (local < RANGE)
    ivec = lanes + i * 16
    plsc.store_scatter(table_v, [local], ivec, mask=msk)
    return 0
  lax.fori_loop(0, IDV, scan_body, 0, unroll=4)
  plsc.subcore_barrier()

  def compact_body(k, cnt):
    ent = table_v[pl.ds(k * 16, 16)]
    m = ent >= 0
    pos = cnt + plsc.cumsum(m.astype(jnp.int32)) - 1
    node = lanes + (k * 16 + base)
    plsc.store_scatter(wb_v, [pos // CHUNK, pos % CHUNK], ent, mask=m)
    plsc.store_scatter(wn_v, [pos // CHUNK, pos % CHUNK], node, mask=m)
    return cnt + jnp.sum(m.astype(jnp.int32))
  m_w = lax.fori_loop(0, TPAD // 16, compact_body, 0)
  plsc.subcore_barrier()

  n_chunks = (m_w + CHUNK - 1) // CHUNK

  @pl.when(m_w > 0)
  def _do_pad():
    wb16 = wb_v[0, pl.ds(0, 16)]
    wn16 = wn_v[0, pl.ds(0, 16)]
    sel0 = (lanes == 0).astype(jnp.int32)
    wb0 = jnp.sum(wb16 * sel0)
    wn0 = jnp.sum(wn16 * sel0)
    padded = n_chunks * CHUNK

    def pad_body(j, _):
      pos = m_w + j * 16 + lanes
      pm = pos < padded
      plsc.store_scatter(wb_v, [pos // CHUNK, pos % CHUNK],
                         jnp.full((16,), 0, jnp.int32) + wb0, mask=pm)
      plsc.store_scatter(wn_v, [pos // CHUNK, pos % CHUNK],
                         jnp.full((16,), 0, jnp.int32) + wn0, mask=pm)
      return 0
    lax.fori_loop(0, CHUNK // 16, pad_body, 0)

  plsc.subcore_barrier()

  @pl.when(m_w > 0)
  def _do_scatter():
    def gather(c, b):
      pltpu.make_async_copy(
          upd_hbm.at[wb_v.at[c]], rows_v.at[b], gsem.at[b]).start()

    def gather_wait(c, b):
      pltpu.make_async_copy(
          upd_hbm.at[wb_v.at[c]], rows_v.at[b], gsem.at[b]).wait()

    def scat(c, b):
      pltpu.make_async_copy(
          rows_v.at[b], out_hbm.at[wn_v.at[c]], ssem.at[b]).start()

    def scat_wait(c, b):
      pltpu.make_async_copy(
          rows_v.at[b], out_hbm.at[wn_v.at[c]], ssem.at[b]).wait()

    gather(0, 0)

    def chunk_body(c, _):
      b = lax.rem(c, 2)
      gather_wait(c, b)
      scat(c, b)

      @pl.when(c + 1 < n_chunks)
      def _():
        @pl.when(c >= 1)
        def _():
          scat_wait(c - 1, 1 - b)
        gather(c + 1, 1 - b)
      return 0
    lax.fori_loop(0, n_chunks, chunk_body, 0)

    @pl.when(n_chunks >= 2)
    def _():
      scat_wait(n_chunks - 2, lax.rem(n_chunks - 2, 2))
    scat_wait(n_chunks - 1, lax.rem(n_chunks - 1, 2))


def _sc_stage(mem128, node_ids, updated):
  mesh = plsc.VectorSubcoreMesh(
      core_axis_name="c", subcore_axis_name="s", num_cores=NC, num_subcores=NS)
  kern = pl_mpmd._mpmd_map(
      [(mesh, _sc_body)],
      [jax.ShapeDtypeStruct((N_NODES, DP), jnp.float32)],
      input_output_aliases={0: 0},
      compiler_params=pltpu.CompilerParams(
          needs_layout_passes=False, use_tc_tiling_on_sc=True),
      scratch_types=[
          pltpu.VMEM((B,), jnp.int32),
          pltpu.VMEM((TPAD,), jnp.int32),
          pltpu.VMEM((NCHUNKS, CHUNK), jnp.int32),
          pltpu.VMEM((NCHUNKS, CHUNK), jnp.int32),
          pltpu.VMEM((2, CHUNK, DP), jnp.float32),
          pltpu.SemaphoreType.DMA,
          pltpu.SemaphoreType.DMA((2,)),
          pltpu.SemaphoreType.DMA((2,)),
      ],
  )
  return kern(mem128, node_ids, updated)[0]


def kernel(mem, embeddings, times, map_centers, node_ids):
  updated, mem128 = _tc_stage(embeddings, map_centers, times, mem)
  out128 = _sc_stage(mem128, node_ids, updated)
  return out128[:, :D]


# speedup vs baseline: 7.7147x; 1.0728x over previous
import jax
import jax.numpy as jnp
from jax import lax
from jax.experimental import pallas as pl
from jax.experimental.pallas import tpu as pltpu
from jax.experimental.pallas import tpu_sc as plsc
from jax._src.pallas import mpmd as pl_mpmd

N_NODES = 100000
D = 64
DP = 128
N_CELLS = 1024
B = 16384


TB = 1024
N_TILES = B // TB
MEMB = 6400


def _tc_body(emb_ref, cent_ref, times_ref, mem_ref, upd_ref, memc_ref):
  emb = emb_ref[...].T
  cent = cent_ref[...].T
  e2 = jnp.sum(emb * emb, axis=1, keepdims=True)
  c2 = jnp.sum(cent * cent, axis=1)[None, :]
  dot = jax.lax.dot_general(
      emb, cent, (((1,), (1,)), ((), ())),
      preferred_element_type=jnp.float32)
  d2 = e2 - 2.0 * dot + c2
  minval = jnp.min(d2, axis=1, keepdims=True)
  iota = lax.broadcasted_iota(jnp.int32, (TB, N_CELLS), 1)
  idx = jnp.min(jnp.where(d2 == minval, iota, N_CELLS), axis=1)
  onehot = (iota == idx[:, None]).astype(jnp.float32)
  gathered = jax.lax.dot_general(
      onehot, cent, (((1,), (0,)), ((), ())),
      preferred_element_type=jnp.float32)
  t = times_ref[0, 0, :]
  w = jax.nn.sigmoid(t)[:, None]
  upd = w * emb + (1.0 - w) * gathered
  zpad = jnp.zeros((TB, DP - D), jnp.float32)
  upd_ref[...] = jnp.concatenate([upd, zpad], axis=1)
  memc_ref[...] = jnp.concatenate(
      [mem_ref[...].T, jnp.zeros((MEMB, DP - D), jnp.float32)], axis=1)


def _tc_stage(embeddings, map_centers, times, mem):
  times3 = times.reshape(N_TILES, 1, TB)
  return pl.pallas_call(
      _tc_body,
      grid=(N_TILES,),
      in_specs=[
          pl.BlockSpec((D, TB), lambda i: (0, i)),
          pl.BlockSpec((D, N_CELLS), lambda i: (0, 0)),
          pl.BlockSpec((1, 1, TB), lambda i: (i, 0, 0)),
          pl.BlockSpec((D, MEMB), lambda i: (0, i)),
      ],
      out_specs=[
          pl.BlockSpec((TB, DP), lambda i: (i, 0)),
          pl.BlockSpec((MEMB, DP), lambda i: (i, 0)),
      ],
      out_shape=[
          jax.ShapeDtypeStruct((B, DP), jnp.float32),
          jax.ShapeDtypeStruct((N_NODES, DP), jnp.float32),
      ],
  )(embeddings.T, map_centers.T, times3, mem.T)


NC = 2
NS = 16
NW = NC * NS
RANGE = 3128
TPAD = ((RANGE + 15) // 16) * 16
CHUNK = 128
NCHUNKS = (RANGE + CHUNK - 1) // CHUNK
IDV = B // 16


def _sc_body(mem_hbm, ids_hbm, upd_hbm, out_hbm,
             ids_v, table_v, wb_v, wn_v, rows_v, sem, gsem, ssem):
  wid = lax.axis_index("s") * NC + lax.axis_index("c")
  base = wid * RANGE

  pltpu.sync_copy(ids_hbm, ids_v)

  lanes = lax.iota(jnp.int32, 16)

  def init_body(k, _):
    table_v[pl.ds(k * 16, 16)] = jnp.full((16,), -1, jnp.int32)
    return 0
  lax.fori_loop(0, TPAD // 16, init_body, 0)

  def scan_body(i, _):
    ids16 = ids_v[pl.ds(i * 16, 16)]
    local = ids16 - base
    msk = (local >= 0) & (local < RANGE)
    ivec = lanes + i * 16
    plsc.store_scatter(table_v, [local], ivec, mask=msk)
    return 0
  lax.fori_loop(0, IDV, scan_body, 0, unroll=4)
  plsc.subcore_barrier()

  def compact_body(k, cnt):
    ent = table_v[pl.ds(k * 16, 16)]
    m = ent >= 0
    pos = cnt + plsc.cumsum(m.astype(jnp.int32)) - 1
    node = lanes + (k * 16 + base)
    plsc.store_scatter(wb_v, [pos // CHUNK, pos % CHUNK], ent, mask=m)
    plsc.store_scatter(wn_v, [pos // CHUNK, pos % CHUNK], node, mask=m)
    return cnt + jnp.sum(m.astype(jnp.int32))
  m_w = lax.fori_loop(0, TPAD // 16, compact_body, 0)
  plsc.subcore_barrier()

  n_chunks = (m_w + CHUNK - 1) // CHUNK

  @pl.when(m_w > 0)
  def _do_pad():
    wb16 = wb_v[0, pl.ds(0, 16)]
    wn16 = wn_v[0, pl.ds(0, 16)]
    sel0 = (lanes == 0).astype(jnp.int32)
    wb0 = jnp.sum(wb16 * sel0)
    wn0 = jnp.sum(wn16 * sel0)
    padded = n_chunks * CHUNK

    def pad_body(j, _):
      pos = m_w + j * 16 + lanes
      pm = pos < padded
      plsc.store_scatter(wb_v, [pos // CHUNK, pos % CHUNK],
                         jnp.full((16,), 0, jnp.int32) + wb0, mask=pm)
      plsc.store_scatter(wn_v, [pos // CHUNK, pos % CHUNK],
                         jnp.full((16,), 0, jnp.int32) + wn0, mask=pm)
      return 0
    lax.fori_loop(0, CHUNK // 16, pad_body, 0)

  plsc.subcore_barrier()

  @pl.when(m_w > 0)
  def _do_scatter():
    def gather(c, b):
      pltpu.make_async_copy(
          upd_hbm.at[wb_v.at[c]], rows_v.at[b], gsem.at[b]).start()

    def gather_wait(c, b):
      pltpu.make_async_copy(
          upd_hbm.at[wb_v.at[c]], rows_v.at[b], gsem.at[b]).wait()

    def scat(c, b):
      pltpu.make_async_copy(
          rows_v.at[b], out_hbm.at[wn_v.at[c]], ssem.at[b]).start()

    def scat_wait(c, b):
      pltpu.make_async_copy(
          rows_v.at[b], out_hbm.at[wn_v.at[c]], ssem.at[b]).wait()

    gather(0, 0)

    def chunk_body(c, _):
      b = lax.rem(c, 2)
      gather_wait(c, b)
      scat(c, b)

      @pl.when(c + 1 < n_chunks)
      def _():
        @pl.when(c >= 1)
        def _():
          scat_wait(c - 1, 1 - b)
        gather(c + 1, 1 - b)
      return 0
    lax.fori_loop(0, n_chunks, chunk_body, 0)

    @pl.when(n_chunks >= 2)
    def _():
      scat_wait(n_chunks - 2, lax.rem(n_chunks - 2, 2))
    scat_wait(n_chunks - 1, lax.rem(n_chunks - 1, 2))


def _sc_stage(mem128, node_ids, updated):
  mesh = plsc.VectorSubcoreMesh(
      core_axis_name="c", subcore_axis_name="s", num_cores=NC, num_subcores=NS)
  kern = pl_mpmd._mpmd_map(
      [(mesh, _sc_body)],
      [jax.ShapeDtypeStruct((N_NODES, DP), jnp.float32)],
      input_output_aliases={0: 0},
      compiler_params=pltpu.CompilerParams(
          needs_layout_passes=False, use_tc_tiling_on_sc=True),
      scratch_types=[
          pltpu.VMEM((B,), jnp.int32),
          pltpu.VMEM((TPAD,), jnp.int32),
          pltpu.VMEM((NCHUNKS, CHUNK), jnp.int32),
          pltpu.VMEM((NCHUNKS, CHUNK), jnp.int32),
          pltpu.VMEM((2, CHUNK, DP), jnp.float32),
          pltpu.SemaphoreType.DMA,
          pltpu.SemaphoreType.DMA((2,)),
          pltpu.SemaphoreType.DMA((2,)),
      ],
  )
  return kern(mem128, node_ids, updated)[0]


def kernel(mem, embeddings, times, map_centers, node_ids):
  updated, mem128 = _tc_stage(embeddings, map_centers, times, mem)
  out128 = _sc_stage(mem128, node_ids, updated)
  return out128[:, :D]
